# baseline (device time: 16002 ns/iter reference)
import jax
import jax.numpy as jnp
from jax import lax
from jax.experimental import pallas as pl
from jax.experimental.pallas import tpu as pltpu

N_DEV = 4


def kernel(x, Wq, Wo, K_ext, V_ext):
    B, Sq, D = x.shape
    _, Skv, Hkv, Dh = K_ext.shape
    Dq = Wq.shape[1]
    Hq_loc = Dq // Dh
    G = Hq_loc // 4
    R = B * Sq
    D2 = D // 2

    x2 = x.reshape(R, D)
    K2 = K_ext.reshape(B * Skv, Hkv * Dh)
    V2 = V_ext.reshape(B * Skv, Hkv * Dh)

    idx = lax.axis_index("i")
    K_loc = lax.dynamic_slice_in_dim(K2, idx * G * Dh, G * Dh, axis=1)
    V_loc = lax.dynamic_slice_in_dim(V2, idx * G * Dh, G * Dh, axis=1)

    def body(x_ref, wq_ref, wo_ref, k_ref, v_ref, out_ref,
             attn_ref, pbuf, rbuf, send_sems, recv_sems):
        my = lax.axis_index("i")
        p1 = my + 1 - 2 * lax.rem(my, 2)
        p2 = (N_DEV - 1) - my

        barrier_sem = pltpu.get_barrier_semaphore()
        for nbr in (p1, p2):
            pl.semaphore_signal(
                barrier_sem, inc=1,
                device_id=(nbr,), device_id_type=pl.DeviceIdType.MESH,
            )

        xb = x_ref[:].astype(jnp.bfloat16)
        wq = wq_ref[:].astype(jnp.bfloat16)
        q = lax.dot_general(xb, wq, (((1,), (0,)), ((), ())),
                            preferred_element_type=jnp.float32)
        q = (q * 0.125).astype(jnp.bfloat16)

        kb = k_ref[:].astype(jnp.bfloat16)
        vb = v_ref[:].astype(jnp.bfloat16)
        for b in range(B):
            rows = slice(b * Sq, (b + 1) * Sq)
            for g in range(G):
                qs = jnp.concatenate(
                    [q[rows, (g * 4 + hh) * Dh:(g * 4 + hh + 1) * Dh]
                     for hh in range(4)], axis=0)
                kg = kb[rows, g * Dh:(g + 1) * Dh]
                vg = vb[rows, g * Dh:(g + 1) * Dh]
                s = lax.dot_general(qs, kg, (((1,), (1,)), ((), ())),
                                    preferred_element_type=jnp.float32)
                p = jnp.exp(s)
                l = jnp.sum(p, axis=1, keepdims=True)
                o = lax.dot_general(p.astype(jnp.bfloat16), vg,
                                    (((1,), (0,)), ((), ())),
                                    preferred_element_type=jnp.float32)
                on = (o / l).astype(jnp.bfloat16)
                for hh in range(4):
                    attn_ref[rows, (g * 4 + hh) * Dh:(g * 4 + hh + 1) * Dh] = (
                        on[hh * Sq:(hh + 1) * Sq, :])

        attn = attn_ref[:]
        wo = wo_ref[:].astype(jnp.bfloat16)

        pl.semaphore_wait(barrier_sem, 2)

        pA = lax.dot_general(attn, wo[:, :D2], (((1,), (0,)), ((), ())),
                             preferred_element_type=jnp.float32)
        pbuf[0] = pA.astype(jnp.bfloat16)
        rA = pltpu.make_async_remote_copy(
            src_ref=pbuf.at[0], dst_ref=rbuf.at[0],
            send_sem=send_sems.at[0], recv_sem=recv_sems.at[0],
            device_id=(p1,), device_id_type=pl.DeviceIdType.MESH,
        )
        rA.start()

        pB = lax.dot_general(attn, wo[:, D2:], (((1,), (0,)), ((), ())),
                             preferred_element_type=jnp.float32)
        pbuf[1] = pB.astype(jnp.bfloat16)
        rB = pltpu.make_async_remote_copy(
            src_ref=pbuf.at[1], dst_ref=rbuf.at[1],
            send_sem=send_sems.at[1], recv_sem=recv_sems.at[1],
            device_id=(p2,), device_id_type=pl.DeviceIdType.MESH,
        )
        rB.start()

        rA.wait_recv()
        sA = pA + rbuf[0].astype(jnp.float32)
        rA.wait_send()
        pbuf[0] = sA.astype(jnp.bfloat16)
        rA2 = pltpu.make_async_remote_copy(
            src_ref=pbuf.at[0], dst_ref=rbuf.at[2],
            send_sem=send_sems.at[2], recv_sem=recv_sems.at[2],
            device_id=(p2,), device_id_type=pl.DeviceIdType.MESH,
        )
        rA2.start()

        rB.wait_recv()
        sB = pB + rbuf[1].astype(jnp.float32)
        rB.wait_send()
        pbuf[1] = sB.astype(jnp.bfloat16)
        rB2 = pltpu.make_async_remote_copy(
            src_ref=pbuf.at[1], dst_ref=rbuf.at[3],
            send_sem=send_sems.at[3], recv_sem=recv_sems.at[3],
            device_id=(p1,), device_id_type=pl.DeviceIdType.MESH,
        )
        rB2.start()

        rA2.wait_recv()
        out_ref[:, :D2] = sA + rbuf[2].astype(jnp.float32)
        rB2.wait_recv()
        out_ref[:, D2:] = sB + rbuf[3].astype(jnp.float32)
        rA2.wait_send()
        rB2.wait_send()

    out2 = pl.pallas_call(
        body,
        out_shape=jax.ShapeDtypeStruct((R, D), jnp.float32),
        in_specs=[pl.BlockSpec(memory_space=pltpu.VMEM)] * 5,
        out_specs=pl.BlockSpec(memory_space=pltpu.VMEM),
        scratch_shapes=[
            pltpu.VMEM((R, Dq), jnp.bfloat16),
            pltpu.VMEM((2, R, D2), jnp.bfloat16),
            pltpu.VMEM((4, R, D2), jnp.bfloat16),
            pltpu.SemaphoreType.DMA((4,)),
            pltpu.SemaphoreType.DMA((4,)),
        ],
        compiler_params=pltpu.CompilerParams(collective_id=0),
    )(x2, Wq, Wo, K_loc, V_loc)
    return out2.reshape(B, Sq, D)


# device time: 13788 ns/iter; 1.1606x vs baseline; 1.1606x over previous
import os

import jax
import jax.numpy as jnp
from jax import lax
from jax.experimental import pallas as pl
from jax.experimental.pallas import tpu as pltpu

N_DEV = 4

_NO_COMM = os.environ.get("KERNEL_NO_COMM") == "1"
_TRIVIAL = os.environ.get("KERNEL_TRIVIAL") == "1"


def kernel(x, Wq, Wo, K_ext, V_ext):
    B, Sq, D = x.shape
    _, Skv, Hkv, Dh = K_ext.shape
    Dq = Wq.shape[1]
    Hq_loc = Dq // Dh
    G = Hq_loc // 4
    R = B * Sq
    D2 = D // 2
    DQ = D // 4

    bf16 = jnp.bfloat16
    xb = x.astype(bf16)
    Wqb = (Wq * 0.125).astype(bf16)
    Wob = Wo.astype(bf16)

    idx = lax.axis_index("i")
    K_loc = lax.dynamic_slice_in_dim(K_ext, idx * G, G, axis=2)
    V_loc = lax.dynamic_slice_in_dim(V_ext, idx * G, G, axis=2)
    K_loc = K_loc.astype(bf16).reshape(B * Skv, G * Dh)
    V_loc = V_loc.astype(bf16).reshape(B * Skv, G * Dh)

    def body(x_ref, wq_ref, wo_ref, k_ref, v_ref, out_ref,
             attn_ref, pbuf, rbuf, send_sems, recv_sems):
        my = lax.axis_index("i")
        p1 = my + 1 - 2 * lax.rem(my, 2)
        p2 = (N_DEV - 1) - my

        if not _NO_COMM:
            barrier_sem = pltpu.get_barrier_semaphore()
            for nbr in (p1, p2):
                pl.semaphore_signal(
                    barrier_sem, inc=1,
                    device_id=(nbr,), device_id_type=pl.DeviceIdType.MESH,
                )

        if _TRIVIAL:
            for b in range(B):
                out_ref[b * Sq:(b + 1) * Sq, :] = x_ref[b]
            return

        wq = wq_ref[:]
        qs_b = [
            lax.dot_general(x_ref[b], wq, (((1,), (0,)), ((), ())),
                            preferred_element_type=jnp.float32
                            ).astype(bf16)
            for b in range(B)
        ]

        kb = k_ref[:]
        vb = v_ref[:]
        for b in range(B):
            rows = slice(b * Sq, (b + 1) * Sq)
            q = qs_b[b]
            for g in range(G):
                qs = jnp.concatenate(
                    [q[:, (g * 4 + hh) * Dh:(g * 4 + hh + 1) * Dh]
                     for hh in range(4)], axis=0)
                kg = kb[rows, g * Dh:(g + 1) * Dh]
                vg = vb[rows, g * Dh:(g + 1) * Dh]
                s = lax.dot_general(qs, kg, (((1,), (1,)), ((), ())),
                                    preferred_element_type=jnp.float32)
                p = jnp.exp(s)
                l = jnp.sum(p, axis=1, keepdims=True)
                o = lax.dot_general(p.astype(bf16), vg,
                                    (((1,), (0,)), ((), ())),
                                    preferred_element_type=jnp.float32)
                on = (o / l).astype(bf16)
                for hh in range(4):
                    attn_ref[rows, (g * 4 + hh) * Dh:(g * 4 + hh + 1) * Dh] = (
                        on[hh * Sq:(hh + 1) * Sq, :])

        attn = attn_ref[:]
        wo = wo_ref[:]

        if not _NO_COMM:
            pl.semaphore_wait(barrier_sem, 2)

        pA = lax.dot_general(attn, wo[:, :D2], (((1,), (0,)), ((), ())),
                             preferred_element_type=jnp.float32)
        if _NO_COMM:
            pB0 = lax.dot_general(attn, wo[:, D2:], (((1,), (0,)), ((), ())),
                                  preferred_element_type=jnp.float32)
            out_ref[:, :D2] = pA.astype(bf16)
            out_ref[:, D2:] = pB0.astype(bf16)
            return

        ph1_partner = (p1, p1, p2, p2)
        ph2_partner = (p2, p2, p1, p1)

        def ph1(c):
            r = pltpu.make_async_remote_copy(
                src_ref=pbuf.at[c], dst_ref=rbuf.at[c],
                send_sem=send_sems.at[c], recv_sem=recv_sems.at[c],
                device_id=(ph1_partner[c],),
                device_id_type=pl.DeviceIdType.MESH,
            )
            r.start()
            return r

        def ph2(c):
            r = pltpu.make_async_remote_copy(
                src_ref=pbuf.at[c], dst_ref=rbuf.at[4 + c],
                send_sem=send_sems.at[4 + c], recv_sem=recv_sems.at[4 + c],
                device_id=(ph2_partner[c],),
                device_id_type=pl.DeviceIdType.MESH,
            )
            r.start()
            return r

        quarters = [None] * 4
        quarters[0] = pA[:, :DQ]
        quarters[1] = pA[:, DQ:]
        pbuf[0] = quarters[0].astype(bf16)
        pbuf[1] = quarters[1].astype(bf16)
        r1 = [None] * 4
        r1[0] = ph1(0)
        r1[1] = ph1(1)

        pB = lax.dot_general(attn, wo[:, D2:], (((1,), (0,)), ((), ())),
                             preferred_element_type=jnp.float32)
        quarters[2] = pB[:, :DQ]
        quarters[3] = pB[:, DQ:]
        pbuf[2] = quarters[2].astype(bf16)
        pbuf[3] = quarters[3].astype(bf16)
        r1[2] = ph1(2)
        r1[3] = ph1(3)

        order = (0, 2, 1, 3)
        sums = [None] * 4
        r2 = [None] * 4
        for c in order:
            r1[c].wait_recv()
            sums[c] = quarters[c] + rbuf[c].astype(jnp.float32)
            r1[c].wait_send()
            pbuf[c] = sums[c].astype(bf16)
            r2[c] = ph2(c)
        for c in order:
            r2[c].wait_recv()
            out_ref[:, c * DQ:(c + 1) * DQ] = (
                sums[c] + rbuf[4 + c].astype(jnp.float32)).astype(bf16)
        for c in order:
            r2[c].wait_send()

    out2 = pl.pallas_call(
        body,
        out_shape=jax.ShapeDtypeStruct((R, D), jnp.bfloat16),
        in_specs=[pl.BlockSpec(memory_space=pltpu.VMEM)] * 5,
        out_specs=pl.BlockSpec(memory_space=pltpu.VMEM),
        scratch_shapes=[
            pltpu.VMEM((R, Dq), jnp.bfloat16),
            pltpu.VMEM((4, R, DQ), jnp.bfloat16),
            pltpu.VMEM((8, R, DQ), jnp.bfloat16),
            pltpu.SemaphoreType.DMA((8,)),
            pltpu.SemaphoreType.DMA((8,)),
        ],
        **({} if _NO_COMM
           else dict(compiler_params=pltpu.CompilerParams(collective_id=0))),
    )(xb, Wqb, Wob, K_loc, V_loc)
    return out2.reshape(B, Sq, D)


# device time: 13739 ns/iter; 1.1647x vs baseline; 1.0036x over previous
import os

import jax
import jax.numpy as jnp
from jax import lax
from jax.experimental import pallas as pl
from jax.experimental.pallas import tpu as pltpu

N_DEV = 4

_NO_COMM = os.environ.get("KERNEL_NO_COMM") == "1"
_TRIVIAL = os.environ.get("KERNEL_TRIVIAL") == "1"


def kernel(x, Wq, Wo, K_ext, V_ext):
    B, Sq, D = x.shape
    _, Skv, Hkv, Dh = K_ext.shape
    Dq = Wq.shape[1]
    Hq_loc = Dq // Dh
    G = Hq_loc // 4
    R = B * Sq
    D2 = D // 2

    bf16 = jnp.bfloat16
    Wqb = (Wq * 0.125).astype(bf16)
    Wob = Wo.astype(bf16)

    idx = lax.axis_index("i")
    K_loc = lax.dynamic_slice_in_dim(K_ext, idx * G, G, axis=2)
    V_loc = lax.dynamic_slice_in_dim(V_ext, idx * G, G, axis=2)
    K_loc = K_loc.astype(bf16).reshape(B * Skv, G * Dh)
    V_loc = V_loc.astype(bf16).reshape(B * Skv, G * Dh)

    def body(x_ref, wq_ref, wo_ref, k_ref, v_ref, out_ref,
             pbuf, rbuf, send_sems, recv_sems):
        my = lax.axis_index("i")
        p1 = my + 1 - 2 * lax.rem(my, 2)
        p2 = (N_DEV - 1) - my

        if not _NO_COMM:
            barrier_sem = pltpu.get_barrier_semaphore()
            for nbr in (p1, p2):
                pl.semaphore_signal(
                    barrier_sem, inc=1,
                    device_id=(nbr,), device_id_type=pl.DeviceIdType.MESH,
                )

        if _TRIVIAL:
            for b in range(B):
                out_ref[b * Sq:(b + 1) * Sq, :] = x_ref[b].astype(bf16)
            return

        wq = wq_ref[:]
        wo = wo_ref[:]
        kb = k_ref[:]
        vb = v_ref[:]

        def ph1(c):
            r = pltpu.make_async_remote_copy(
                src_ref=pbuf.at[c], dst_ref=rbuf.at[c],
                send_sem=send_sems.at[c], recv_sem=recv_sems.at[c],
                device_id=((p1, p2)[c % 2],),
                device_id_type=pl.DeviceIdType.MESH,
            )
            r.start()
            return r

        def ph2(c):
            r = pltpu.make_async_remote_copy(
                src_ref=pbuf.at[c], dst_ref=rbuf.at[4 + c],
                send_sem=send_sems.at[4 + c], recv_sem=recv_sems.at[4 + c],
                device_id=((p2, p1)[c % 2],),
                device_id_type=pl.DeviceIdType.MESH,
            )
            r.start()
            return r

        chunks = [None] * 4
        r1 = [None] * 4
        for b in range(B):
            rows = slice(b * Sq, (b + 1) * Sq)
            q = lax.dot_general(x_ref[b].astype(bf16), wq,
                                (((1,), (0,)), ((), ())),
                                preferred_element_type=jnp.float32
                                ).astype(bf16)

            attn_cols = []
            for g in range(G):
                qs = jnp.concatenate(
                    [q[:, (g * 4 + hh) * Dh:(g * 4 + hh + 1) * Dh]
                     for hh in range(4)], axis=0)
                kg = kb[rows, g * Dh:(g + 1) * Dh]
                vg = vb[rows, g * Dh:(g + 1) * Dh]
                s = lax.dot_general(qs, kg, (((1,), (1,)), ((), ())),
                                    preferred_element_type=jnp.float32)
                p = jnp.exp(s)
                l = jnp.sum(p, axis=1, keepdims=True)
                o = lax.dot_general(p.astype(bf16), vg,
                                    (((1,), (0,)), ((), ())),
                                    preferred_element_type=jnp.float32)
                on = (o / l).astype(bf16)
                attn_cols.extend(
                    on[hh * Sq:(hh + 1) * Sq, :] for hh in range(4))
            attn_b = jnp.concatenate(attn_cols, axis=1)

            if _NO_COMM:
                pb = lax.dot_general(attn_b, wo, (((1,), (0,)), ((), ())),
                                     preferred_element_type=jnp.float32)
                out_ref[rows, :] = pb.astype(bf16)
                continue

            if b == 0:
                pl.semaphore_wait(barrier_sem, 2)

            for half in range(2):
                c = 2 * b + half
                pc = lax.dot_general(
                    attn_b, wo[:, half * D2:(half + 1) * D2],
                    (((1,), (0,)), ((), ())),
                    preferred_element_type=jnp.float32)
                chunks[c] = pc
                pbuf[c] = pc.astype(bf16)
                r1[c] = ph1(c)

        if _NO_COMM:
            return

        sums = [None] * 4
        r2 = [None] * 4
        for c in range(4):
            r1[c].wait_recv()
            sums[c] = chunks[c] + rbuf[c].astype(jnp.float32)
            r1[c].wait_send()
            pbuf[c] = sums[c].astype(bf16)
            r2[c] = ph2(c)
        for c in range(4):
            b, half = divmod(c, 2)
            r2[c].wait_recv()
            out_ref[b * Sq:(b + 1) * Sq, half * D2:(half + 1) * D2] = (
                sums[c] + rbuf[4 + c].astype(jnp.float32)).astype(bf16)
        for c in range(4):
            r2[c].wait_send()

    out2 = pl.pallas_call(
        body,
        out_shape=jax.ShapeDtypeStruct((R, D), jnp.bfloat16),
        in_specs=[pl.BlockSpec(memory_space=pltpu.VMEM)] * 5,
        out_specs=pl.BlockSpec(memory_space=pltpu.VMEM),
        scratch_shapes=[
            pltpu.VMEM((4, Sq, D2), jnp.bfloat16),
            pltpu.VMEM((8, Sq, D2), jnp.bfloat16),
            pltpu.SemaphoreType.DMA((8,)),
            pltpu.SemaphoreType.DMA((8,)),
        ],
        **({} if _NO_COMM
           else dict(compiler_params=pltpu.CompilerParams(collective_id=0))),
    )(x, Wqb, Wob, K_loc, V_loc)
    return out2.reshape(B, Sq, D)
